# baseline (speedup 1.0000x reference)
import functools

import jax
import jax.numpy as jnp
from jax.experimental import pallas as pl
from jax.experimental.pallas import tpu as pltpu

T = 100
DEPTH = 10
NPT = 2 ** DEPTH - 1
H = 128
IN = 128
B = 20


def _sigmoid_half(y):
    return 0.5 + 0.5 * jnp.tanh(y)


def _dot_t(a, w):
    return jax.lax.dot_general(a, w, (((1,), (1,)), ((), ())),
                               preferred_element_type=jnp.float32)


def _tree_gru_kernel(x_hbm, wih_ref, whh_ref, bgh_ref, bxn_ref, out_ref,
                     x_scr, h_scr, sem):
    i = pl.program_id(0)

    NTOP = 2 ** (DEPTH - 1) - 1

    def start_copies(prog, slot):
        for t in range(B):
            pltpu.make_async_copy(
                x_hbm.at[pl.ds((prog * B + t) * NPT, NTOP), :],
                x_scr.at[slot, t, 0:NTOP], sem.at[slot, 0]).start()
        for t in range(B):
            pltpu.make_async_copy(
                x_hbm.at[pl.ds((prog * B + t) * NPT + NTOP, NPT - NTOP), :],
                x_scr.at[slot, t, NTOP:NPT], sem.at[slot, 1]).start()

    @pl.when(i == 0)
    def _():
        start_copies(0, 0)

    @pl.when(i + 1 < pl.num_programs(0))
    def _():
        start_copies(i + 1, (i + 1) % 2)

    slot = i % 2
    for t in range(B):
        pltpu.make_async_copy(
            x_hbm.at[pl.ds(t * NPT, NTOP), :],
            x_scr.at[slot, t, 0:NTOP], sem.at[slot, 0]).wait()
    x_cur = x_scr.at[slot]

    wih = wih_ref[...]
    whh = whh_ref[...]
    bgh = bgh_ref[...]
    bxn = bxn_ref[...]

    x0 = x_cur[:, 0, :]
    gx = _dot_t(x0, wih)
    r = _sigmoid_half(gx[:, :H] + bgh[:, :H])
    z = _sigmoid_half(gx[:, H:2 * H] + bgh[:, H:2 * H])
    n = jnp.tanh(gx[:, 2 * H:] + bxn + r * bgh[:, 2 * H:])
    h_scr[:, 0:1, :] = ((1.0 - z) * n).reshape(B, 1, H)

    pooled = None
    for l in range(1, DEPTH):
        m = 2 ** (l - 1)
        nl = 2 ** l
        hp = h_scr[:, m - 1:2 * m - 1, :].reshape(B * m, H)
        gh = _dot_t(hp, whh) + bgh
        if l == DEPTH - 1:
          for t in range(B):
              pltpu.make_async_copy(
                  x_hbm.at[pl.ds(t * NPT + NTOP, NPT - NTOP), :],
                  x_scr.at[slot, t, NTOP:NPT], sem.at[slot, 1]).wait()
        halves = []
        for s in (0, 1):
          x = x_cur[:, nl - 1 + s:2 * nl - 1:2, :].reshape(B * m, IN)
          gx = _dot_t(x, wih)
          r = _sigmoid_half(gx[:, :H] + gh[:, :H])
          z = _sigmoid_half(gx[:, H:2 * H] + gh[:, H:2 * H])
          n = jnp.tanh(gx[:, 2 * H:] + bxn + r * gh[:, 2 * H:])
          halves.append(n + z * (hp - n))
        if l < DEPTH - 1:
          h_scr[:, nl - 1:2 * nl - 1:2, :] = halves[0].reshape(B, m, H)
          h_scr[:, nl:2 * nl - 1:2, :] = halves[1].reshape(B, m, H)
        else:
          mL = jnp.max(halves[0].reshape(B, m, H), axis=1)
          mR = jnp.max(halves[1].reshape(B, m, H), axis=1)
          pooled = jnp.maximum(mL, mR)

    out_ref[0] = pooled


@functools.partial(jax.jit, static_argnames=())
def kernel(inputs, W_ih, W_hh, b_ih, b_hh, parent):
    del parent
    gate_scale = jnp.concatenate(
        [jnp.full((2 * H, 1), 0.5, jnp.float32),
         jnp.ones((H, 1), jnp.float32)], axis=0)
    wih_t = W_ih * gate_scale
    whh_t = W_hh * gate_scale
    bgh = jnp.concatenate(
        [0.5 * (b_ih[:2 * H] + b_hh[:2 * H]), b_hh[2 * H:]]).reshape(1, 3 * H)
    bxn = b_ih[2 * H:].reshape(1, H)

    grid = (T // B,)
    return pl.pallas_call(
        _tree_gru_kernel,
        grid=grid,
        in_specs=[
            pl.BlockSpec(memory_space=pl.ANY),
            pl.BlockSpec((3 * H, IN), lambda i: (0, 0)),
            pl.BlockSpec((3 * H, H), lambda i: (0, 0)),
            pl.BlockSpec((1, 3 * H), lambda i: (0, 0)),
            pl.BlockSpec((1, H), lambda i: (0, 0)),
        ],
        out_specs=pl.BlockSpec((1, B, H), lambda i: (i, 0, 0)),
        out_shape=jax.ShapeDtypeStruct((T // B, B, H), jnp.float32),
        scratch_shapes=[
            pltpu.VMEM((2, B, NPT, IN), jnp.float32),
            pltpu.VMEM((B, NPT, H), jnp.float32),
            pltpu.SemaphoreType.DMA((2, 2)),
        ],
        compiler_params=pltpu.CompilerParams(
            dimension_semantics=("arbitrary",),
        ),
    )(inputs, wih_t, whh_t, bgh, bxn).reshape(T, H)


# speedup vs baseline: 1.0329x; 1.0329x over previous
import functools

import jax
import jax.numpy as jnp
from jax.experimental import pallas as pl
from jax.experimental.pallas import tpu as pltpu

T = 100
DEPTH = 10
NPT = 2 ** DEPTH - 1
H = 128
IN = 128
B = 25


def _sigmoid_half(y):
    return 0.5 + 0.5 * jnp.tanh(y)


def _dot_t(a, w):
    return jax.lax.dot_general(a, w, (((1,), (1,)), ((), ())),
                               preferred_element_type=jnp.float32)


def _tree_gru_kernel(x_hbm, wih_ref, whh_ref, bgh_ref, bxn_ref, out_ref,
                     x_scr, h_scr, sem):
    i = pl.program_id(0)

    NTOP = 2 ** (DEPTH - 1) - 1

    def start_copies(prog, slot):
        for t in range(B):
            pltpu.make_async_copy(
                x_hbm.at[pl.ds((prog * B + t) * NPT, NTOP), :],
                x_scr.at[slot, t, 0:NTOP], sem.at[slot, 0]).start()
        for t in range(B):
            pltpu.make_async_copy(
                x_hbm.at[pl.ds((prog * B + t) * NPT + NTOP, NPT - NTOP), :],
                x_scr.at[slot, t, NTOP:NPT], sem.at[slot, 1]).start()

    @pl.when(i == 0)
    def _():
        start_copies(0, 0)

    @pl.when(i + 1 < pl.num_programs(0))
    def _():
        start_copies(i + 1, (i + 1) % 2)

    slot = i % 2
    for t in range(B):
        pltpu.make_async_copy(
            x_hbm.at[pl.ds(t * NPT, NTOP), :],
            x_scr.at[slot, t, 0:NTOP], sem.at[slot, 0]).wait()
    x_cur = x_scr.at[slot]

    wih = wih_ref[...]
    whh = whh_ref[...]
    bgh = bgh_ref[...]
    bxn = bxn_ref[...]

    x0 = x_cur[:, 0, :]
    gx = _dot_t(x0, wih)
    r = _sigmoid_half(gx[:, :H] + bgh[:, :H])
    z = _sigmoid_half(gx[:, H:2 * H] + bgh[:, H:2 * H])
    n = jnp.tanh(gx[:, 2 * H:] + bxn + r * bgh[:, 2 * H:])
    h_scr[:, 0:1, :] = ((1.0 - z) * n).reshape(B, 1, H)

    pooled = None
    for l in range(1, DEPTH):
        m = 2 ** (l - 1)
        nl = 2 ** l
        hp = h_scr[:, m - 1:2 * m - 1, :].reshape(B * m, H)
        gh = _dot_t(hp, whh) + bgh
        if l == DEPTH - 1:
          for t in range(B):
              pltpu.make_async_copy(
                  x_hbm.at[pl.ds(t * NPT + NTOP, NPT - NTOP), :],
                  x_scr.at[slot, t, NTOP:NPT], sem.at[slot, 1]).wait()
        halves = []
        for s in (0, 1):
          x = x_cur[:, nl - 1 + s:2 * nl - 1:2, :].reshape(B * m, IN)
          gx = _dot_t(x, wih)
          r = _sigmoid_half(gx[:, :H] + gh[:, :H])
          z = _sigmoid_half(gx[:, H:2 * H] + gh[:, H:2 * H])
          n = jnp.tanh(gx[:, 2 * H:] + bxn + r * gh[:, 2 * H:])
          halves.append(n + z * (hp - n))
        if l < DEPTH - 1:
          h_scr[:, nl - 1:2 * nl - 1:2, :] = halves[0].reshape(B, m, H)
          h_scr[:, nl:2 * nl - 1:2, :] = halves[1].reshape(B, m, H)
        else:
          mL = jnp.max(halves[0].reshape(B, m, H), axis=1)
          mR = jnp.max(halves[1].reshape(B, m, H), axis=1)
          pooled = jnp.maximum(mL, mR)

    out_ref[0] = pooled


@functools.partial(jax.jit, static_argnames=())
def kernel(inputs, W_ih, W_hh, b_ih, b_hh, parent):
    del parent
    gate_scale = jnp.concatenate(
        [jnp.full((2 * H, 1), 0.5, jnp.float32),
         jnp.ones((H, 1), jnp.float32)], axis=0)
    wih_t = W_ih * gate_scale
    whh_t = W_hh * gate_scale
    bgh = jnp.concatenate(
        [0.5 * (b_ih[:2 * H] + b_hh[:2 * H]), b_hh[2 * H:]]).reshape(1, 3 * H)
    bxn = b_ih[2 * H:].reshape(1, H)

    grid = (T // B,)
    return pl.pallas_call(
        _tree_gru_kernel,
        grid=grid,
        in_specs=[
            pl.BlockSpec(memory_space=pl.ANY),
            pl.BlockSpec((3 * H, IN), lambda i: (0, 0)),
            pl.BlockSpec((3 * H, H), lambda i: (0, 0)),
            pl.BlockSpec((1, 3 * H), lambda i: (0, 0)),
            pl.BlockSpec((1, H), lambda i: (0, 0)),
        ],
        out_specs=pl.BlockSpec((1, B, H), lambda i: (i, 0, 0)),
        out_shape=jax.ShapeDtypeStruct((T // B, B, H), jnp.float32),
        scratch_shapes=[
            pltpu.VMEM((2, B, NPT, IN), jnp.float32),
            pltpu.VMEM((B, NPT, H), jnp.float32),
            pltpu.SemaphoreType.DMA((2, 2)),
        ],
        compiler_params=pltpu.CompilerParams(
            dimension_semantics=("arbitrary",),
        ),
    )(inputs, wih_t, whh_t, bgh, bxn).reshape(T, H)
